# baseline (speedup 1.0000x reference)
import functools

import jax
import jax.numpy as jnp
from jax import lax
from jax.experimental import pallas as pl
from jax.experimental.pallas import tpu as pltpu
from jax.experimental.pallas import tpu_sc as plsc

NUM_ROWS = 8192
DIM = 256
BATCH = 32 * 1024
NUM_CORES = 2
NUM_SUBCORES = 16
NUM_WORKERS = NUM_CORES * NUM_SUBCORES
B_PER_W = BATCH // NUM_WORKERS
CHUNK = 128
NCHUNKS = B_PER_W // CHUNK
NBUF = 3


@functools.partial(
    pl.kernel,
    out_type=jax.ShapeDtypeStruct((BATCH, DIM), jnp.float32),
    mesh=plsc.VectorSubcoreMesh(core_axis_name="c", subcore_axis_name="s"),
    scratch_types=(
        [pltpu.VMEM((NCHUNKS, CHUNK), jnp.int32)]
        + [pltpu.VMEM((CHUNK, DIM), jnp.float32) for _ in range(NBUF)]
        + [pltpu.SemaphoreType.DMA for _ in range(2 * NBUF)]
    ),
)
def _gather_call(idx_hbm, table_hbm, out_hbm, idx_v, *bufs_and_sems):
    bufs = bufs_and_sems[:NBUF]
    gsems = bufs_and_sems[NBUF:2 * NBUF]
    wsems = bufs_and_sems[2 * NBUF:]
    wid = lax.axis_index("s") * NUM_CORES + lax.axis_index("c")
    base = wid * B_PER_W
    for c in range(NCHUNKS):
        pltpu.sync_copy(idx_hbm.at[wid, pl.ds(c * CHUNK, CHUNK)], idx_v.at[c])

    def gather(c, b):
        return pltpu.async_copy(table_hbm.at[idx_v.at[c]], bufs[b], gsems[b])

    def write(c, b):
        return pltpu.async_copy(
            bufs[b], out_hbm.at[pl.ds(base + c * CHUNK, CHUNK)], wsems[b])

    for b in range(NBUF):
        gather(b, b)
    def body(c, _):
        b = lax.rem(c, NBUF)
        for bs in range(NBUF):
            @pl.when(b == bs)
            def _():
                gather_done = pltpu.make_async_copy(
                    table_hbm.at[idx_v.at[c]], bufs[bs], gsems[bs])
                gather_done.wait()
                write(c, bs).wait()
                gather(c + NBUF, bs)
        return ()

    lax.fori_loop(0, NCHUNKS - NBUF, body, (), unroll=False)
    for c in range(NCHUNKS - NBUF, NCHUNKS):
        b = c % NBUF
        pltpu.make_async_copy(
            table_hbm.at[idx_v.at[c]], bufs[b], gsems[b]).wait()
        write(c, b).wait()


def kernel(embed_id, weight):
    out = _gather_call(embed_id, weight)
    return out.reshape(embed_id.shape + (weight.shape[-1],))
